# baseline (speedup 1.0000x reference)
import functools
import numpy as np
import jax
import jax.numpy as jnp
from jax import lax
from jax.experimental import pallas as pl
from jax.experimental.pallas import tpu as pltpu
from jax.experimental.pallas import tpu_sc as plsc

D_MODEL = 1024
NUM_SETS = 64
K_TOP = 8
NEG = -1e30
BLK = 512

_SC_NUM_CORES = 2
_SC_NUM_SUBCORES = 16
_NW = _SC_NUM_CORES * _SC_NUM_SUBCORES


def _scores_body(x_ref, desc_ref, wq_ref, bq_ref, s_ref):
    scale = 1.0 / np.sqrt(D_MODEL)
    q = jax.lax.dot_general(
        x_ref[0], wq_ref[...], (((1,), (1,)), ((), ())),
        preferred_element_type=jnp.float32) + bq_ref[...]
    s_ref[0] = jax.lax.dot_general(
        q, desc_ref[0], (((1,), (1,)), ((), ())),
        preferred_element_type=jnp.float32) * scale


def _combine_body(w_ref, set_ref, out_ref):
    out_ref[0] = jax.lax.dot_general(
        w_ref[...], set_ref[0], (((1,), (0,)), ((), ())),
        preferred_element_type=jnp.float32)


def _route_body(s_hbm, tts_hbm, w_hbm, s_v, tts_v, wout_v, win_v, per_w):
    wid = lax.axis_index("s") * _SC_NUM_CORES + lax.axis_index("c")
    base = wid * per_w
    pltpu.sync_copy(s_hbm.at[pl.ds(base * NUM_SETS, per_w * NUM_SETS)], s_v)
    pltpu.sync_copy(tts_hbm.at[pl.ds(base * 16, per_w * 16)], tts_v)

    lanes = lax.iota(jnp.int32, 16)

    def token(t, _):
        idx = plsc.load_gather(tts_v, [t * 16 + lanes])
        plsc.store_scatter(win_v, [idx], lanes)
        keep = plsc.load_gather(win_v, [idx]) == lanes
        s = plsc.load_gather(s_v, [t * NUM_SETS + idx])
        s_m = jnp.where(keep, s, NEG)
        sk, sv = plsc.sort_key_val(s_m, idx, descending=True)
        top8 = lanes < K_TOP
        s8 = jnp.where(top8, sk, NEG)
        m = jnp.max(s8)
        e = jnp.exp(s8 - m)
        w = e / jnp.sum(e)
        valid = top8 & (sk > NEG * 0.5)
        zeros = jnp.zeros((16,), jnp.float32)
        for jj in range(4):
            plsc.store_scatter(wout_v, [t * NUM_SETS + lanes + (16 * jj)],
                               zeros)
        plsc.store_scatter(wout_v, [t * NUM_SETS + sv], w, mask=valid)
        return 0

    lax.fori_loop(0, per_w, token, 0)
    pltpu.sync_copy(wout_v, w_hbm.at[pl.ds(base * NUM_SETS,
                                           per_w * NUM_SETS)])


@jax.jit
def _run(token_states, set_states, desc_router, tts2, W_q, b_q2):
    batch, seq_len, d = token_states.shape
    nb = seq_len // BLK
    per_w = seq_len // _NW

    mesh = plsc.VectorSubcoreMesh(core_axis_name="c", subcore_axis_name="s",
                                  num_cores=_SC_NUM_CORES,
                                  num_subcores=_SC_NUM_SUBCORES)

    def scores_b(b):
        return pl.pallas_call(
            _scores_body,
            grid=(nb,),
            in_specs=[
                pl.BlockSpec((1, BLK, d), lambda i: (0, i, 0)),
                pl.BlockSpec((1, NUM_SETS, d), lambda i: (0, 0, 0)),
                pl.BlockSpec((d, d), lambda i: (0, 0)),
                pl.BlockSpec((1, d), lambda i: (0, 0)),
            ],
            out_specs=pl.BlockSpec((1, BLK, NUM_SETS), lambda i: (0, i, 0)),
            out_shape=jax.ShapeDtypeStruct((1, seq_len, NUM_SETS),
                                           jnp.float32),
        )(token_states[b:b + 1], desc_router[b:b + 1], W_q, b_q2)

    def route_b(scores, b):
        return pl.kernel(
            functools.partial(_route_body, per_w=per_w),
            out_type=jax.ShapeDtypeStruct((seq_len * NUM_SETS,),
                                          jnp.float32),
            mesh=mesh,
            compiler_params=pltpu.CompilerParams(needs_layout_passes=False),
            scratch_types=[
                pltpu.VMEM((per_w * NUM_SETS,), jnp.float32),
                pltpu.VMEM((per_w * 16,), jnp.int32),
                pltpu.VMEM((per_w * NUM_SETS,), jnp.float32),
                pltpu.VMEM((NUM_SETS,), jnp.int32),
            ],
        )(scores.reshape(seq_len * NUM_SETS),
          tts2[b * seq_len:(b + 1) * seq_len].reshape(-1))

    def combine_b(weights, b):
        return pl.pallas_call(
            _combine_body,
            grid=(nb,),
            in_specs=[
                pl.BlockSpec((BLK, NUM_SETS), lambda i: (i, 0)),
                pl.BlockSpec((1, NUM_SETS, d), lambda i: (0, 0, 0)),
            ],
            out_specs=pl.BlockSpec((1, BLK, d), lambda i: (0, i, 0)),
            out_shape=jax.ShapeDtypeStruct((1, seq_len, d), jnp.float32),
        )(weights.reshape(seq_len, NUM_SETS), set_states[b:b + 1])

    s0 = scores_b(0)
    w0 = route_b(s0, 0)
    s1 = scores_b(1)
    w1 = route_b(s1, 1)
    o0 = combine_b(w0, 0)
    o1 = combine_b(w1, 1)
    return jnp.concatenate([o0, o1], axis=0)


def kernel(token_states, set_states, desc_router, token_to_sets, W_q, b_q):
    batch = token_states.shape[0]
    tts = token_to_sets.astype(jnp.int32)
    tts2 = jnp.concatenate([tts] * batch, axis=0)
    return _run(token_states, set_states, desc_router, tts2, W_q,
                b_q.reshape(1, -1))


# speedup vs baseline: 1.3842x; 1.3842x over previous
import functools
import numpy as np
import jax
import jax.numpy as jnp
from jax import lax
from jax.experimental import pallas as pl
from jax.experimental.pallas import tpu as pltpu
from jax.experimental.pallas import tpu_sc as plsc

D_MODEL = 1024
NUM_SETS = 64
K_TOP = 8
NEG = -1e30
BLK = 512

_SC_NUM_CORES = 2
_SC_NUM_SUBCORES = 16
_NW = _SC_NUM_CORES * _SC_NUM_SUBCORES


def _scores_body(x_ref, desc_ref, wq_ref, bq_ref, s_ref):
    scale = 1.0 / np.sqrt(D_MODEL)
    q = jax.lax.dot_general(
        x_ref[0], wq_ref[...], (((1,), (1,)), ((), ())),
        preferred_element_type=jnp.float32) + bq_ref[...]
    s_ref[0] = jax.lax.dot_general(
        q, desc_ref[0], (((1,), (1,)), ((), ())),
        preferred_element_type=jnp.float32) * scale


def _combine_body(w_ref, set_ref, out_ref):
    out_ref[0] = jax.lax.dot_general(
        w_ref[...], set_ref[0], (((1,), (0,)), ((), ())),
        preferred_element_type=jnp.float32)


def _route_body(s_hbm, tts_hbm, w_hbm, s_v, tts_v, wout_v, win_v, per_w):
    wid = lax.axis_index("s") * _SC_NUM_CORES + lax.axis_index("c")
    base = wid * per_w
    pltpu.sync_copy(s_hbm.at[pl.ds(base * NUM_SETS, per_w * NUM_SETS)], s_v)
    pltpu.sync_copy(tts_hbm.at[pl.ds(base * 16, per_w * 16)], tts_v)

    lanes = lax.iota(jnp.int32, 16)
    zeros = jnp.zeros((16,), jnp.float32)

    def token(t, _):
        idx = plsc.load_gather(tts_v, [t * 16 + lanes])
        plsc.store_scatter(win_v, [idx], lanes)
        keep = plsc.load_gather(win_v, [idx]) == lanes
        s = plsc.load_gather(s_v, [t * NUM_SETS + idx])
        s_m = jnp.where(keep, s, NEG)
        sk, sv = plsc.sort_key_val(s_m, idx, descending=True)
        top8 = lanes < K_TOP
        s8 = jnp.where(top8, sk, NEG)
        m = jnp.max(s8)
        e = jnp.exp(s8 - m)
        w = e / jnp.sum(e)
        valid = top8 & (sk > NEG * 0.5)
        for jj in range(4):
            wout_v[pl.ds(t * NUM_SETS + 16 * jj, 16)] = zeros
        plsc.store_scatter(wout_v, [t * NUM_SETS + sv], w, mask=valid)
        return 0

    lax.fori_loop(0, per_w, token, 0, unroll=4)
    pltpu.sync_copy(wout_v, w_hbm.at[pl.ds(base * NUM_SETS,
                                           per_w * NUM_SETS)])


@jax.jit
def _run(token_states, set_states, desc_router, tts2, W_q, b_q2):
    batch, seq_len, d = token_states.shape
    nb = seq_len // BLK
    tokens = batch * seq_len
    per_w = tokens // _NW

    scores = pl.pallas_call(
        _scores_body,
        grid=(batch, nb),
        in_specs=[
            pl.BlockSpec((1, BLK, d), lambda b, i: (b, i, 0)),
            pl.BlockSpec((1, NUM_SETS, d), lambda b, i: (b, 0, 0)),
            pl.BlockSpec((d, d), lambda b, i: (0, 0)),
            pl.BlockSpec((1, d), lambda b, i: (0, 0)),
        ],
        out_specs=pl.BlockSpec((1, BLK, NUM_SETS), lambda b, i: (b, i, 0)),
        out_shape=jax.ShapeDtypeStruct((batch, seq_len, NUM_SETS),
                                       jnp.float32),
    )(token_states, desc_router, W_q, b_q2)
    scores_f = scores.reshape(tokens * NUM_SETS)

    mesh = plsc.VectorSubcoreMesh(core_axis_name="c", subcore_axis_name="s",
                                  num_cores=_SC_NUM_CORES,
                                  num_subcores=_SC_NUM_SUBCORES)
    weights = pl.kernel(
        functools.partial(_route_body, per_w=per_w),
        out_type=jax.ShapeDtypeStruct((tokens * NUM_SETS,), jnp.float32),
        mesh=mesh,
        compiler_params=pltpu.CompilerParams(needs_layout_passes=False),
        scratch_types=[
            pltpu.VMEM((per_w * NUM_SETS,), jnp.float32),
            pltpu.VMEM((per_w * 16,), jnp.int32),
            pltpu.VMEM((per_w * NUM_SETS,), jnp.float32),
            pltpu.VMEM((NUM_SETS,), jnp.int32),
        ],
    )(scores_f, tts2.reshape(-1))
    weights = weights.reshape(tokens, NUM_SETS)

    out = pl.pallas_call(
        _combine_body,
        grid=(batch, nb),
        in_specs=[
            pl.BlockSpec((BLK, NUM_SETS),
                         lambda b, i, nb=nb: (b * nb + i, 0)),
            pl.BlockSpec((1, NUM_SETS, d), lambda b, i: (b, 0, 0)),
        ],
        out_specs=pl.BlockSpec((1, BLK, d), lambda b, i: (b, i, 0)),
        out_shape=jax.ShapeDtypeStruct((batch, seq_len, d), jnp.float32),
    )(weights, set_states)
    return out


def kernel(token_states, set_states, desc_router, token_to_sets, W_q, b_q):
    batch = token_states.shape[0]
    tts = token_to_sets.astype(jnp.int32)
    tts2 = jnp.concatenate([tts] * batch, axis=0)
    return _run(token_states, set_states, desc_router, tts2, W_q,
                b_q.reshape(1, -1))


# speedup vs baseline: 1.4035x; 1.0139x over previous
import functools
import numpy as np
import jax
import jax.numpy as jnp
from jax import lax
from jax.experimental import pallas as pl
from jax.experimental.pallas import tpu as pltpu
from jax.experimental.pallas import tpu_sc as plsc

D_MODEL = 1024
NUM_SETS = 64
K_TOP = 8
NEG = -1e30
BLK = 512

_SC_NUM_CORES = 2
_SC_NUM_SUBCORES = 16
_NW = _SC_NUM_CORES * _SC_NUM_SUBCORES


def _scores_body(x_ref, desc_ref, wq_ref, bq_ref, s_ref):
    scale = 1.0 / np.sqrt(D_MODEL)
    q = jax.lax.dot_general(
        x_ref[0], wq_ref[...], (((1,), (1,)), ((), ())),
        preferred_element_type=jnp.float32) + bq_ref[...]
    s_ref[0] = jax.lax.dot_general(
        q, desc_ref[0], (((1,), (1,)), ((), ())),
        preferred_element_type=jnp.float32) * scale


def _combine_body(w_ref, set_ref, out_ref):
    out_ref[0] = jax.lax.dot_general(
        w_ref[...], set_ref[0], (((1,), (0,)), ((), ())),
        preferred_element_type=jnp.float32)


_NCHUNK = 4


def _route_body(s_hbm, tts_hbm, w_hbm, s_v, tts_v, wout_v, win_v,
                sem_t, sem_i, sem_o, per_w):
    wid = lax.axis_index("s") * _SC_NUM_CORES + lax.axis_index("c")
    base = wid * per_w
    ch = per_w // _NCHUNK

    h_t = pltpu.async_copy(tts_hbm.at[pl.ds(base * 16, per_w * 16)],
                           tts_v, sem_t)
    h_in = [
        pltpu.async_copy(
            s_hbm.at[pl.ds((base + c * ch) * NUM_SETS, ch * NUM_SETS)],
            s_v.at[pl.ds(c * ch * NUM_SETS, ch * NUM_SETS)], sem_i)
        for c in range(_NCHUNK)
    ]

    lanes = lax.iota(jnp.int32, 16)
    zeros = jnp.zeros((16,), jnp.float32)

    def token(t, _):
        idx = plsc.load_gather(tts_v, [t * 16 + lanes])
        plsc.store_scatter(win_v, [idx], lanes)
        keep = plsc.load_gather(win_v, [idx]) == lanes
        s = plsc.load_gather(s_v, [t * NUM_SETS + idx])
        s_m = jnp.where(keep, s, NEG)
        sk, sv = plsc.sort_key_val(s_m, idx, descending=True)
        top8 = lanes < K_TOP
        s8 = jnp.where(top8, sk, NEG)
        m = jnp.max(s8)
        e = jnp.exp(s8 - m)
        w = e / jnp.sum(e)
        valid = top8 & (sk > NEG * 0.5)
        for jj in range(4):
            wout_v[pl.ds(t * NUM_SETS + 16 * jj, 16)] = zeros
        plsc.store_scatter(wout_v, [t * NUM_SETS + sv], w, mask=valid)
        return 0

    h_t.wait()
    h_out = []
    for c in range(_NCHUNK):
        h_in[c].wait()
        lax.fori_loop(c * ch, (c + 1) * ch, token, 0, unroll=4)
        h_out.append(pltpu.async_copy(
            wout_v.at[pl.ds(c * ch * NUM_SETS, ch * NUM_SETS)],
            w_hbm.at[pl.ds((base + c * ch) * NUM_SETS, ch * NUM_SETS)],
            sem_o))
    for h in h_out:
        h.wait()


@jax.jit
def _run(token_states, set_states, desc_router, tts2, W_q, b_q2):
    batch, seq_len, d = token_states.shape
    nb = seq_len // BLK
    tokens = batch * seq_len
    per_w = tokens // _NW

    scores = pl.pallas_call(
        _scores_body,
        grid=(batch, nb),
        in_specs=[
            pl.BlockSpec((1, BLK, d), lambda b, i: (b, i, 0)),
            pl.BlockSpec((1, NUM_SETS, d), lambda b, i: (b, 0, 0)),
            pl.BlockSpec((d, d), lambda b, i: (0, 0)),
            pl.BlockSpec((1, d), lambda b, i: (0, 0)),
        ],
        out_specs=pl.BlockSpec((1, BLK, NUM_SETS), lambda b, i: (b, i, 0)),
        out_shape=jax.ShapeDtypeStruct((batch, seq_len, NUM_SETS),
                                       jnp.float32),
    )(token_states, desc_router, W_q, b_q2)
    scores_f = scores.reshape(tokens * NUM_SETS)

    mesh = plsc.VectorSubcoreMesh(core_axis_name="c", subcore_axis_name="s",
                                  num_cores=_SC_NUM_CORES,
                                  num_subcores=_SC_NUM_SUBCORES)
    weights = pl.kernel(
        functools.partial(_route_body, per_w=per_w),
        out_type=jax.ShapeDtypeStruct((tokens * NUM_SETS,), jnp.float32),
        mesh=mesh,
        compiler_params=pltpu.CompilerParams(needs_layout_passes=False),
        scratch_types=[
            pltpu.VMEM((per_w * NUM_SETS,), jnp.float32),
            pltpu.VMEM((per_w * 16,), jnp.int32),
            pltpu.VMEM((per_w * NUM_SETS,), jnp.float32),
            pltpu.VMEM((NUM_SETS,), jnp.int32),
            pltpu.SemaphoreType.DMA,
            pltpu.SemaphoreType.DMA,
            pltpu.SemaphoreType.DMA,
        ],
    )(scores_f, tts2.reshape(-1))
    weights = weights.reshape(tokens, NUM_SETS)

    out = pl.pallas_call(
        _combine_body,
        grid=(batch, nb),
        in_specs=[
            pl.BlockSpec((BLK, NUM_SETS),
                         lambda b, i, nb=nb: (b * nb + i, 0)),
            pl.BlockSpec((1, NUM_SETS, d), lambda b, i: (b, 0, 0)),
        ],
        out_specs=pl.BlockSpec((1, BLK, d), lambda b, i: (b, i, 0)),
        out_shape=jax.ShapeDtypeStruct((batch, seq_len, d), jnp.float32),
    )(weights, set_states)
    return out


def kernel(token_states, set_states, desc_router, token_to_sets, W_q, b_q):
    batch = token_states.shape[0]
    tts = token_to_sets.astype(jnp.int32)
    tts2 = jnp.concatenate([tts] * batch, axis=0)
    return _run(token_states, set_states, desc_router, tts2, W_q,
                b_q.reshape(1, -1))
